# Initial kernel scaffold; baseline (speedup 1.0000x reference)
import jax
import jax.numpy as jnp
from jax.experimental import pallas as pl


def kernel(inputs, edge_index, a1_W, a1_b, a2_W, a2_b, c1_W, c1_b, c2_W, c2_b, c3_W, c3_b, cls_W1, cls_b1, cls_W2, cls_b2, box_W1, box_b1, lora_A, lora_B, e_W1, e_b1, e_W2, e_b2, e_W3, e_b3):
    raise NotImplementedError("write your pallas kernel here")


# speedup vs baseline: 5.1116x; 5.1116x over previous
import jax
import jax.numpy as jnp
from jax import lax
from jax.experimental import pallas as pl
from jax.experimental.pallas import tpu as pltpu
from jax.experimental.pallas import tpu_sc as plsc

N = 10000
D = 128
E = 320000
NPAD = 10240
NW = 32
CH = 128
CPT = 79
EPT = CPT * CH
EPAD = NW * EPT
RPT = NPAD // 16
BR = 1024
BE = 2048
ZPOS = 50.0


def _pe_table():
    inv_freq = 1.0 / (55 * 10) ** (jnp.arange(0, D, 2, dtype=jnp.float32) / D)
    t = jnp.arange(0, 55, dtype=jnp.float32)[:, None]
    ang = t * inv_freq[None, :]
    pe = jnp.concatenate([jnp.sin(ang), jnp.cos(ang)], axis=1)
    return jnp.pad(pe, ((0, 64 - 55), (0, 0)))


def _deg_body(dst_hbm, zer_hbm, one_hbm, out_hbm, idx_v, ones_v, buf_v, deg_sh):
    c = lax.axis_index("c")
    s = lax.axis_index("s")
    wid = c * 16 + s
    pltpu.sync_copy(zer_hbm, deg_sh.at[pl.ds(s * RPT, RPT)])
    pltpu.sync_copy(one_hbm, ones_v)
    plsc.subcore_barrier()

    def chunk(j, carry):
        base = wid * EPT + j * CH
        pltpu.sync_copy(dst_hbm.at[pl.ds(base, CH)], idx_v)
        pltpu.sync_copy(ones_v, deg_sh.at[idx_v], add=True)
        return carry

    lax.fori_loop(0, CPT, chunk, 0)
    plsc.subcore_barrier()

    def rb(k, carry):
        r = s * RPT + k * CH
        pltpu.sync_copy(deg_sh.at[pl.ds(r, CH)], buf_v)
        pltpu.sync_copy(buf_v, out_hbm.at[pl.ds(c * NPAD + r, CH)])
        return carry

    lax.fori_loop(0, RPT // CH, rb, 0)


def _conv_body(y_hbm, src_hbm, dst_hbm, zer_hbm, out_hbm, si_v, di_v, rows_v,
               acc_sh):
    c = lax.axis_index("c")
    s = lax.axis_index("s")
    wid = c * 16 + s
    pltpu.sync_copy(zer_hbm, acc_sh.at[pl.ds(s * RPT, RPT)])
    plsc.subcore_barrier()

    def chunk(j, carry):
        base = wid * EPT + j * CH
        pltpu.sync_copy(src_hbm.at[pl.ds(base, CH)], si_v)
        pltpu.sync_copy(dst_hbm.at[pl.ds(base, CH)], di_v)
        pltpu.sync_copy(y_hbm.at[si_v], rows_v)
        pltpu.sync_copy(rows_v, acc_sh.at[di_v], add=True)
        return carry

    lax.fori_loop(0, CPT, chunk, 0)
    plsc.subcore_barrier()

    def rb(k, carry):
        r = s * RPT + k * CH
        pltpu.sync_copy(acc_sh.at[pl.ds(r, CH)], rows_v)
        pltpu.sync_copy(rows_v, out_hbm.at[pl.ds(c * NPAD + r, CH)])
        return carry

    lax.fori_loop(0, RPT // CH, rb, 0)


def _edge_body(xs_hbm, xd_hbm, src_hbm, dst_hbm, out_hbm, si_v, di_v, rows_v):
    wid = lax.axis_index("c") * 16 + lax.axis_index("s")

    def chunk(j, carry):
        base = wid * EPT + j * CH
        pltpu.sync_copy(src_hbm.at[pl.ds(base, CH)], si_v)
        pltpu.sync_copy(dst_hbm.at[pl.ds(base, CH)], di_v)
        pltpu.sync_copy(xs_hbm.at[si_v], rows_v)
        pltpu.sync_copy(xd_hbm.at[di_v], rows_v, add=True)
        pltpu.sync_copy(rows_v, out_hbm.at[pl.ds(base, CH)])
        return carry

    lax.fori_loop(0, CPT, chunk, 0)


def _a_body(inp_ref, degp_ref, pe_ref, a1w_ref, a1b_ref, a2w_ref, a2b_ref,
            c1w_ref, y1_ref, dis_ref):
    xb = inp_ref[...]
    t = jnp.maximum(jnp.dot(xb, a1w_ref[...],
                            preferred_element_type=jnp.float32) + a1b_ref[...],
                    0.0)
    f = jnp.dot(t, a2w_ref[...], preferred_element_type=jnp.float32) \
        + a2b_ref[...]
    pos = (xb[:, 0:1] * ZPOS).astype(jnp.int32)
    iot = lax.broadcasted_iota(jnp.int32, (1, 64), 1)
    oh = (pos == iot).astype(jnp.float32)
    x0 = f + jnp.dot(oh, pe_ref[...], preferred_element_type=jnp.float32)
    dp = degp_ref[...]
    deg = dp[0, :, 0:1] + dp[1, :, 0:1] + 1.0
    dis = lax.rsqrt(deg)
    dis_ref[...] = dis
    y1_ref[...] = dis * jnp.dot(x0, c1w_ref[...],
                                preferred_element_type=jnp.float32)


def _c_body(accp_ref, y_ref, dis_ref, b_ref, w_ref, yout_ref):
    ap = accp_ref[...]
    dis = dis_ref[...]
    x = jnp.maximum(dis * (ap[0] + ap[1] + y_ref[...]) + b_ref[...], 0.0)
    yout_ref[...] = dis * jnp.dot(x, w_ref[...],
                                  preferred_element_type=jnp.float32)


def _c4_body(accp_ref, y_ref, dis_ref, inp4_ref, c3b_ref, clsw1_ref,
             clsb1_ref, clsw2_ref, clsb2_ref, boxw1_ref, boxb1_ref, la_ref,
             lb_ref, ew1a_ref, ew1b_ref, eb1_ref, x3_ref, pred_ref, box_ref,
             xs_ref, xd_ref):
    ap = accp_ref[...]
    dis = dis_ref[...]
    x3 = dis * (ap[0] + ap[1] + y_ref[...]) + c3b_ref[...]
    x3_ref[...] = x3
    p = jnp.maximum(jnp.dot(x3, clsw1_ref[...],
                            preferred_element_type=jnp.float32)
                    + clsb1_ref[...], 0.0)
    pred_ref[...] = jnp.dot(p, clsw2_ref[...],
                            preferred_element_type=jnp.float32) + clsb2_ref[...]
    h = jnp.maximum(jnp.dot(x3, boxw1_ref[...],
                            preferred_element_type=jnp.float32)
                    + boxb1_ref[...], 0.0)
    ha = jnp.dot(h, la_ref[...], preferred_element_type=jnp.float32)
    hb = jnp.dot(ha, lb_ref[...], preferred_element_type=jnp.float32)
    box_ref[...] = jnp.tanh(hb[:, 0:4]) + inp4_ref[...]
    xs_ref[...] = jnp.dot(x3, ew1a_ref[...],
                          preferred_element_type=jnp.float32) + eb1_ref[...]
    xd_ref[...] = jnp.dot(x3, ew1b_ref[...],
                          preferred_element_type=jnp.float32)


def _f_body(h_ref, w2_ref, b2_ref, w3_ref, b3_ref, o_ref):
    h = jnp.maximum(h_ref[...], 0.0)
    h = jnp.maximum(jnp.dot(h, w2_ref[...],
                            preferred_element_type=jnp.float32) + b2_ref[...],
                    0.0)
    o_ref[...] = jax.nn.sigmoid(
        jnp.dot(h, w3_ref[...], preferred_element_type=jnp.float32)
        + b3_ref[...])


def _full(shape):
    return pl.BlockSpec(shape, lambda i: tuple(0 for _ in shape))


def kernel(inputs, edge_index, a1_W, a1_b, a2_W, a2_b, c1_W, c1_b, c2_W, c2_b,
           c3_W, c3_b, cls_W1, cls_b1, cls_W2, cls_b2, box_W1, box_b1, lora_A,
           lora_B, e_W1, e_b1, e_W2, e_b2, e_W3, e_b3):
    f32 = jnp.float32
    pe = _pe_table()
    inp_p = jnp.pad(inputs, ((0, NPAD - N), (0, 0)))
    src_p = jnp.pad(edge_index[0], (0, EPAD - E), constant_values=N)
    dst_p = jnp.pad(edge_index[1], (0, EPAD - E), constant_values=N)
    zer_d = jnp.zeros((RPT, D), f32)
    one_d = jnp.ones((CH, D), f32)

    mesh = plsc.VectorSubcoreMesh(core_axis_name="c", subcore_axis_name="s")

    deg_call = pl.kernel(
        _deg_body,
        out_type=jax.ShapeDtypeStruct((2 * NPAD, D), f32),
        mesh=mesh,
        scratch_types=[
            pltpu.VMEM((CH,), jnp.int32),
            pltpu.VMEM((CH, D), f32),
            pltpu.VMEM((CH, D), f32),
            pltpu.VMEM_SHARED((NPAD, D), f32),
        ],
    )
    degp = deg_call(dst_p, zer_d, one_d).reshape(2, NPAD, D)

    grid = NPAD // BR
    y1, dis = pl.pallas_call(
        _a_body,
        grid=(grid,),
        in_specs=[
            pl.BlockSpec((BR, D), lambda i: (i, 0)),
            pl.BlockSpec((2, BR, D), lambda i: (0, i, 0)),
            _full((64, D)), _full((D, D)), _full((1, D)),
            _full((D, D)), _full((1, D)), _full((D, D)),
        ],
        out_specs=[pl.BlockSpec((BR, D), lambda i: (i, 0)),
                   pl.BlockSpec((BR, 1), lambda i: (i, 0))],
        out_shape=[jax.ShapeDtypeStruct((NPAD, D), f32),
                   jax.ShapeDtypeStruct((NPAD, 1), f32)],
    )(inp_p, degp, pe, a1_W, a1_b.reshape(1, D), a2_W, a2_b.reshape(1, D),
      c1_W)

    conv_call = pl.kernel(
        _conv_body,
        out_type=jax.ShapeDtypeStruct((2 * NPAD, D), f32),
        mesh=mesh,
        scratch_types=[
            pltpu.VMEM((CH,), jnp.int32),
            pltpu.VMEM((CH,), jnp.int32),
            pltpu.VMEM((CH, D), f32),
            pltpu.VMEM_SHARED((NPAD, D), f32),
        ],
    )

    def conv_epilogue(accp, y, b, w):
        return pl.pallas_call(
            _c_body,
            grid=(grid,),
            in_specs=[
                pl.BlockSpec((2, BR, D), lambda i: (0, i, 0)),
                pl.BlockSpec((BR, D), lambda i: (i, 0)),
                pl.BlockSpec((BR, 1), lambda i: (i, 0)),
                _full((1, D)), _full((D, D)),
            ],
            out_specs=pl.BlockSpec((BR, D), lambda i: (i, 0)),
            out_shape=jax.ShapeDtypeStruct((NPAD, D), f32),
        )(accp, y, dis, b.reshape(1, D), w)

    accp1 = conv_call(y1, src_p, dst_p, zer_d).reshape(2, NPAD, D)
    y2 = conv_epilogue(accp1, y1, c1_b, c2_W)
    accp2 = conv_call(y2, src_p, dst_p, zer_d).reshape(2, NPAD, D)
    y3 = conv_epilogue(accp2, y2, c2_b, c3_W)
    accp3 = conv_call(y3, src_p, dst_p, zer_d).reshape(2, NPAD, D)

    lap = jnp.pad(lora_A, ((0, 0), (0, 4)))
    lbp = jnp.pad(lora_B, ((0, 4), (0, 4)))
    x3, pred, box, xs1, xd1 = pl.pallas_call(
        _c4_body,
        grid=(grid,),
        in_specs=[
            pl.BlockSpec((2, BR, D), lambda i: (0, i, 0)),
            pl.BlockSpec((BR, D), lambda i: (i, 0)),
            pl.BlockSpec((BR, 1), lambda i: (i, 0)),
            pl.BlockSpec((BR, 4), lambda i: (i, 0)),
            _full((1, D)),
            _full((D, D // 2)), _full((1, D // 2)),
            _full((D // 2, 16)), _full((1, 16)),
            _full((D, D // 2)), _full((1, D // 2)),
            _full((D // 2, 8)), _full((8, 8)),
            _full((D, D)), _full((D, D)), _full((1, D)),
        ],
        out_specs=[pl.BlockSpec((BR, D), lambda i: (i, 0)),
                   pl.BlockSpec((BR, 16), lambda i: (i, 0)),
                   pl.BlockSpec((BR, 4), lambda i: (i, 0)),
                   pl.BlockSpec((BR, D), lambda i: (i, 0)),
                   pl.BlockSpec((BR, D), lambda i: (i, 0))],
        out_shape=[jax.ShapeDtypeStruct((NPAD, D), f32),
                   jax.ShapeDtypeStruct((NPAD, 16), f32),
                   jax.ShapeDtypeStruct((NPAD, 4), f32),
                   jax.ShapeDtypeStruct((NPAD, D), f32),
                   jax.ShapeDtypeStruct((NPAD, D), f32)],
    )(accp3, y3, dis, inp_p[:, 1:5], c3_b.reshape(1, D), cls_W1,
      cls_b1.reshape(1, D // 2), cls_W2, cls_b2.reshape(1, 16), box_W1,
      box_b1.reshape(1, D // 2), lap, lbp, e_W1[:D], e_W1[D:],
      e_b1.reshape(1, D))

    edge_call = pl.kernel(
        _edge_body,
        out_type=jax.ShapeDtypeStruct((EPAD, D), f32),
        mesh=mesh,
        scratch_types=[
            pltpu.VMEM((CH,), jnp.int32),
            pltpu.VMEM((CH,), jnp.int32),
            pltpu.VMEM((CH, D), f32),
        ],
    )
    h1 = edge_call(xs1, xd1, src_p, dst_p)

    edge_full = pl.pallas_call(
        _f_body,
        grid=(EPAD // BE,),
        in_specs=[
            pl.BlockSpec((BE, D), lambda i: (i, 0)),
            _full((D, D // 2)), _full((1, D // 2)),
            _full((D // 2, 8)), _full((1, 8)),
        ],
        out_specs=pl.BlockSpec((BE, 8), lambda i: (i, 0)),
        out_shape=jax.ShapeDtypeStruct((EPAD, 8), f32),
    )(h1, e_W2, e_b2.reshape(1, D // 2), jnp.pad(e_W3, ((0, 0), (0, 7))),
      jnp.pad(e_b3.reshape(1, 1), ((0, 0), (0, 7))))

    return (pred[:N], box[:N], edge_full[:E, 0:1], x3[:N])
